# baseline (speedup 1.0000x reference)
import functools

import jax
import jax.numpy as jnp
from jax import lax
from jax.experimental import pallas as pl
from jax.experimental.pallas import tpu as pltpu
from jax.experimental.pallas import tpu_sc as plsc

N = 16384
EMBED = 64
HIDDEN = 128

_NC = 2
_NS = 16
_NW = _NC * _NS
_ROWS_W = N // _NW
_CHUNK = 128
_NCHUNK = _ROWS_W // _CHUNK


def _sc_gather_body(idx_hbm, table_hbm, x_hbm, idx_v, rows_v, sem):
    wid = lax.axis_index("s") * _NC + lax.axis_index("c")
    for t in range(3):
        pltpu.sync_copy(idx_hbm.at[t * _NW + wid], idx_v)
        cps = [
            pltpu.async_copy(
                table_hbm.at[idx_v.at[j]],
                rows_v.at[pl.ds(j * _CHUNK, _CHUNK)],
                sem,
            )
            for j in range(_NCHUNK)
        ]
        for c in cps:
            c.wait()
        pltpu.sync_copy(rows_v, x_hbm.at[t, pl.ds(wid * _ROWS_W, _ROWS_W)])


def _sc_gather(idx_all, table):
    mesh = plsc.VectorSubcoreMesh(core_axis_name="c", subcore_axis_name="s")
    kern = functools.partial(
        pl.kernel,
        mesh=mesh,
        out_type=jax.ShapeDtypeStruct((3, N, EMBED), jnp.float32),
        scratch_types=[
            pltpu.VMEM((_NCHUNK, _CHUNK), jnp.int32),
            pltpu.VMEM((_ROWS_W, EMBED), jnp.float32),
            pltpu.SemaphoreType.DMA,
        ],
        compiler_params=pltpu.CompilerParams(use_tc_tiling_on_sc=False),
    )(_sc_gather_body)
    return kern(idx_all, table)


_TBLK = 2048


def _tt_body(tt_ref, eye_ref, out_ref):
    out_ref[...] = jax.lax.dot_general(
        tt_ref[...], eye_ref[...],
        (((0,), (0,)), ((), ())),
        preferred_element_type=jnp.float32,
    )


def _tc_transpose(table_t, eye):
    num_rows = table_t.shape[1]
    return pl.pallas_call(
        _tt_body,
        grid=(num_rows // _TBLK,),
        in_specs=[
            pl.BlockSpec((EMBED, _TBLK), lambda i: (0, i)),
            pl.BlockSpec((EMBED, EMBED), lambda i: (0, 0)),
        ],
        out_specs=pl.BlockSpec((_TBLK, EMBED), lambda i: (i, 0)),
        out_shape=jax.ShapeDtypeStruct((num_rows, EMBED), jnp.float32),
    )(table_t, eye)


_BLK = 512


def _tc_project_body(x_ref, w_ref, b_ref, o_ref):
    w = w_ref[...]
    bb = b_ref[...]
    for t in range(3):
        yt = jnp.dot(x_ref[t], w, preferred_element_type=jnp.float32)
        o_ref[:, t * HIDDEN:(t + 1) * HIDDEN] = yt + bb


def _tc_project(x3, wt, b2):
    return pl.pallas_call(
        _tc_project_body,
        grid=(N // _BLK,),
        in_specs=[
            pl.BlockSpec((3, _BLK, EMBED), lambda i: (0, i, 0)),
            pl.BlockSpec((EMBED, HIDDEN), lambda i: (0, 0)),
            pl.BlockSpec((1, HIDDEN), lambda i: (0, 0)),
        ],
        out_specs=pl.BlockSpec((_BLK, 3 * HIDDEN), lambda i: (i, 0)),
        out_shape=jax.ShapeDtypeStruct((N, 3 * HIDDEN), jnp.float32),
    )(x3, wt, b2)


def kernel(s, r, o, table, W, b):
    idx_all = (
        jnp.stack([s, r, o])
        .astype(jnp.int32)
        .reshape(3 * _NW, _NCHUNK, _CHUNK)
    )
    table_rm = _tc_transpose(table.T, jnp.eye(EMBED, dtype=jnp.float32))
    x3 = _sc_gather(idx_all, table_rm)
    wt = W.T
    b2 = b.reshape(1, HIDDEN)
    return _tc_project(x3, wt, b2)


# speedup vs baseline: 1.0035x; 1.0035x over previous
import functools

import jax
import jax.numpy as jnp
from jax import lax
from jax.experimental import pallas as pl
from jax.experimental.pallas import tpu as pltpu
from jax.experimental.pallas import tpu_sc as plsc

N = 16384
EMBED = 64
HIDDEN = 128

_NC = 2
_NS = 16
_NW = _NC * _NS
_ROWS_W = N // _NW
_CHUNK = 128
_NCHUNK = _ROWS_W // _CHUNK


def _sc_gather_body(idx_hbm, table_hbm, x_hbm, idx_v, rows_v, sem):
    wid = lax.axis_index("s") * _NC + lax.axis_index("c")
    for t in range(3):
        pltpu.sync_copy(idx_hbm.at[t * _NW + wid], idx_v)
        cps = [
            pltpu.async_copy(
                table_hbm.at[idx_v.at[j]],
                rows_v.at[pl.ds(j * _CHUNK, _CHUNK)],
                sem,
            )
            for j in range(_NCHUNK)
        ]
        for c in cps:
            c.wait()
        pltpu.sync_copy(rows_v, x_hbm.at[t, pl.ds(wid * _ROWS_W, _ROWS_W)])


def _sc_gather(idx_all, table):
    mesh = plsc.VectorSubcoreMesh(core_axis_name="c", subcore_axis_name="s")
    kern = functools.partial(
        pl.kernel,
        mesh=mesh,
        out_type=jax.ShapeDtypeStruct((3, N, EMBED), jnp.float32),
        scratch_types=[
            pltpu.VMEM((_NCHUNK, _CHUNK), jnp.int32),
            pltpu.VMEM((_ROWS_W, EMBED), jnp.float32),
            pltpu.SemaphoreType.DMA,
        ],
        compiler_params=pltpu.CompilerParams(use_tc_tiling_on_sc=False),
    )(_sc_gather_body)
    return kern(idx_all, table)


_TBLK = 2048


def _tt_body(tt_ref, eye_ref, out_ref):
    out_ref[...] = jax.lax.dot_general(
        tt_ref[...], eye_ref[...],
        (((0,), (0,)), ((), ())),
        preferred_element_type=jnp.float32,
    )


def _tc_transpose(table_t, eye):
    num_rows = table_t.shape[1]
    return pl.pallas_call(
        _tt_body,
        grid=(pl.cdiv(num_rows, _TBLK),),
        in_specs=[
            pl.BlockSpec((EMBED, _TBLK), lambda i: (0, i)),
            pl.BlockSpec((EMBED, EMBED), lambda i: (0, 0)),
        ],
        out_specs=pl.BlockSpec((_TBLK, EMBED), lambda i: (i, 0)),
        out_shape=jax.ShapeDtypeStruct((num_rows, EMBED), jnp.float32),
    )(table_t, eye)


_BLK = 512


def _tc_project_body(x_ref, w_ref, b_ref, o_ref):
    w = w_ref[...]
    bb = b_ref[...]
    for t in range(3):
        yt = jnp.dot(x_ref[t], w, preferred_element_type=jnp.float32)
        o_ref[:, t * HIDDEN:(t + 1) * HIDDEN] = yt + bb


def _tc_project(x3, wt, b2):
    return pl.pallas_call(
        _tc_project_body,
        grid=(N // _BLK,),
        in_specs=[
            pl.BlockSpec((3, _BLK, EMBED), lambda i: (0, i, 0)),
            pl.BlockSpec((EMBED, HIDDEN), lambda i: (0, 0)),
            pl.BlockSpec((1, HIDDEN), lambda i: (0, 0)),
        ],
        out_specs=pl.BlockSpec((_BLK, 3 * HIDDEN), lambda i: (i, 0)),
        out_shape=jax.ShapeDtypeStruct((N, 3 * HIDDEN), jnp.float32),
    )(x3, wt, b2)


def kernel(s, r, o, table, W, b):
    idx_all = (
        jnp.stack([s, r, o])
        .astype(jnp.int32)
        .reshape(3 * _NW, _NCHUNK, _CHUNK)
    )
    table_rm = _tc_transpose(table.T, jnp.eye(EMBED, dtype=jnp.float32))
    x3 = _sc_gather(idx_all, table_rm)
    wt = W.T
    b2 = b.reshape(1, HIDDEN)
    return _tc_project(x3, wt, b2)


# speedup vs baseline: 2.4406x; 2.4320x over previous
import functools

import jax
import jax.numpy as jnp
from jax import lax
from jax.experimental import pallas as pl
from jax.experimental.pallas import tpu as pltpu
from jax.experimental.pallas import tpu_sc as plsc

N = 16384
EMBED = 64
HIDDEN = 128
NROWS = 1000000

_NC = 2
_NS = 16
_NW = _NC * _NS
_ROWS_W = N // _NW
_CHUNK = 128
_NCHUNK = _ROWS_W // _CHUNK

_BP = 8192
_NB = pl.cdiv(NROWS, _BP)


def _tt_body(tt_ref, eye_ref, out_ref):
    t = jax.lax.dot_general(
        tt_ref[...], eye_ref[...],
        (((0,), (0,)), ((), ())),
        preferred_element_type=jnp.float32,
    )
    out_ref[0, :, 0:EMBED] = t
    out_ref[0, :, EMBED:2 * EMBED] = t


def _tc_transpose_dup(table_t, eye):
    return pl.pallas_call(
        _tt_body,
        grid=(_NB,),
        in_specs=[
            pl.BlockSpec((EMBED, _BP), lambda i: (0, i)),
            pl.BlockSpec((EMBED, EMBED), lambda i: (0, 0)),
        ],
        out_specs=pl.BlockSpec((1, _BP, 2 * EMBED), lambda i: (i, 0, 0)),
        out_shape=jax.ShapeDtypeStruct((_NB, _BP, 2 * EMBED), jnp.float32),
    )(table_t, eye)


def _sc_gather_body(idx_hbm, table_hbm, x_hbm, idx_v, rows_v, sem):
    wid = lax.axis_index("s") * _NC + lax.axis_index("c")
    for t in range(3):
        pltpu.sync_copy(idx_hbm.at[t * _NW + wid], idx_v)
        cps = [
            pltpu.async_copy(
                table_hbm.at[idx_v.at[j]],
                rows_v.at[pl.ds(j * _CHUNK, _CHUNK)],
                sem,
            )
            for j in range(_NCHUNK)
        ]
        for c in cps:
            c.wait()
        pltpu.sync_copy(rows_v, x_hbm.at[t, pl.ds(wid * _ROWS_W, _ROWS_W)])


def _sc_gather(idx_all, table_pairs):
    mesh = plsc.VectorSubcoreMesh(core_axis_name="c", subcore_axis_name="s")
    kern = functools.partial(
        pl.kernel,
        mesh=mesh,
        out_type=jax.ShapeDtypeStruct((3, N, 2 * EMBED), jnp.float32),
        scratch_types=[
            pltpu.VMEM((_NCHUNK, _CHUNK), jnp.int32),
            pltpu.VMEM((_ROWS_W, 2 * EMBED), jnp.float32),
            pltpu.SemaphoreType.DMA,
        ],
        compiler_params=pltpu.CompilerParams(use_tc_tiling_on_sc=False),
    )(_sc_gather_body)
    return kern(idx_all, table_pairs)


_BLK = 512


def _tc_project_body(x_ref, w_ref, b_ref, o_ref):
    w = w_ref[...]
    bb = b_ref[...]
    for t in range(3):
        xt = x_ref[t][:, 0:EMBED]
        yt = jnp.dot(xt, w, preferred_element_type=jnp.float32)
        o_ref[:, t * HIDDEN:(t + 1) * HIDDEN] = yt + bb


def _tc_project(x3, wt, b2):
    return pl.pallas_call(
        _tc_project_body,
        grid=(N // _BLK,),
        in_specs=[
            pl.BlockSpec((3, _BLK, 2 * EMBED), lambda i: (0, i, 0)),
            pl.BlockSpec((EMBED, HIDDEN), lambda i: (0, 0)),
            pl.BlockSpec((1, HIDDEN), lambda i: (0, 0)),
        ],
        out_specs=pl.BlockSpec((_BLK, 3 * HIDDEN), lambda i: (i, 0)),
        out_shape=jax.ShapeDtypeStruct((N, 3 * HIDDEN), jnp.float32),
    )(x3, wt, b2)


def kernel(s, r, o, table, W, b):
    idx_all = (
        jnp.stack([s, r, o])
        .astype(jnp.int32)
        .reshape(3 * _NW, _NCHUNK, _CHUNK)
    )
    tbl_pairs = _tc_transpose_dup(table.T, jnp.eye(EMBED, dtype=jnp.float32))
    tbl_pairs = tbl_pairs.reshape(_NB * _BP, 2 * EMBED)
    x3 = _sc_gather(idx_all, tbl_pairs)
    wt = W.T
    b2 = b.reshape(1, HIDDEN)
    return _tc_project(x3, wt, b2)
